# Initial kernel scaffold; baseline (speedup 1.0000x reference)
import jax
import jax.numpy as jnp
from jax.experimental import pallas as pl


def kernel(inputs, table, pos_enc):
    raise NotImplementedError("write your pallas kernel here")


# speedup vs baseline: 2.3317x; 2.3317x over previous
import jax
import jax.numpy as jnp
from jax import lax
from jax.experimental import pallas as pl
from jax.experimental.pallas import tpu as pltpu
from jax.experimental.pallas import tpu_sc as plsc

SEQ = 200
DIM = 128
NCORES = 2
NSUB = 16
NW = NCORES * NSUB
ROWS = 1024 * SEQ
RPW = ROWS // NW
CHUNK = 40
NCH = RPW // CHUNK
NBUF = 4
SCALE = 11.313708498984761


def _body(idx_hbm, table_hbm, pos_hbm, out_hbm,
          idx_v, pos_v,
          in0, in1, in2, in3, ot0, ot1, ot2, ot3,
          g0, g1, g2, g3, o0, o1, o2, o3):
    ins = (in0, in1, in2, in3)
    outs = (ot0, ot1, ot2, ot3)
    gsems = (g0, g1, g2, g3)
    osems = (o0, o1, o2, o3)

    wid = lax.axis_index("s") * NCORES + lax.axis_index("c")
    row0 = wid * RPW

    pltpu.sync_copy(idx_hbm.at[pl.ds(wid * NCH, NCH)], idx_v)
    pltpu.sync_copy(pos_hbm, pos_v)

    def gather_start(c, b):
        pltpu.async_copy(table_hbm.at[idx_v.at[c]], ins[b], gsems[b])

    def gather_wait(c, b):
        pltpu.make_async_copy(table_hbm.at[idx_v.at[c]], ins[b], gsems[b]).wait()

    def out_start(c, b):
        pltpu.async_copy(outs[b], out_hbm.at[pl.ds(row0 + c * CHUNK, CHUNK)],
                         osems[b])

    def out_wait(c, b):
        pltpu.make_async_copy(outs[b],
                              out_hbm.at[pl.ds(row0 + c * CHUNK, CHUNK)],
                              osems[b]).wait()

    def compute(c, b):
        off = lax.rem(c, 5) * CHUNK
        src = ins[b]
        dst = outs[b]

        def row_fma(r, _):
            p = off + r
            for k in range(DIM // 16):
                sl = pl.ds(k * 16, 16)
                dst[r, sl] = src[r, sl] * SCALE + pos_v[p, sl]
            return _

        lax.fori_loop(0, CHUNK, row_fma, 0, unroll=2)

    for b in range(NBUF):
        gather_start(b, b)

    for b in range(NBUF):
        gather_wait(b, b)
        compute(b, b)
        gather_start(b + NBUF, b)
        out_start(b, b)

    def cycle(t, _):
        for b in range(NBUF):
            c = t * NBUF + b
            gather_wait(c, b)
            out_wait(c - NBUF, b)
            compute(c, b)
            gather_start(c + NBUF, b)
            out_start(c, b)
        return _

    lax.fori_loop(1, NCH // NBUF - 1, cycle, 0)

    for b in range(NBUF):
        c = NCH - NBUF + b
        gather_wait(c, b)
        out_wait(c - NBUF, b)
        compute(c, b)
        out_start(c, b)
    for b in range(NBUF):
        out_wait(NCH - NBUF + b, b)


def kernel(inputs, table, pos_enc):
    flat_idx = inputs.reshape(ROWS // CHUNK, CHUNK).astype(jnp.int32)

    mesh = plsc.VectorSubcoreMesh(core_axis_name="c", subcore_axis_name="s")
    run = pl.kernel(
        _body,
        mesh=mesh,
        out_type=jax.ShapeDtypeStruct((ROWS, DIM), jnp.float32),
        scratch_types=[
            pltpu.VMEM((NCH, CHUNK), jnp.int32),
            pltpu.VMEM((SEQ, DIM), jnp.float32),
        ] + [pltpu.VMEM((CHUNK, DIM), jnp.float32)] * (2 * NBUF)
          + [pltpu.SemaphoreType.DMA] * (2 * NBUF),
    )
    out = run(flat_idx, table, pos_enc)
    return out.reshape(1024, SEQ, DIM)


# speedup vs baseline: 6.9372x; 2.9752x over previous
import jax
import jax.numpy as jnp
from jax import lax
from jax.experimental import pallas as pl
from jax.experimental.pallas import tpu as pltpu
from jax.experimental.pallas import tpu_sc as plsc

SEQ = 200
DIM = 128
NCORES = 2
NSUB = 16
NW = NCORES * NSUB
ROWS = 1024 * SEQ
RPW = ROWS // NW
CHUNK = 40
NCH = RPW // CHUNK
NBUF = 4
SCALE = 11.313708498984761


def _body(idx_hbm, table_hbm, pos_hbm, out_hbm,
          idx_v, pos_v,
          in0, in1, in2, in3, ot0, ot1, ot2, ot3,
          g0, g1, g2, g3, o0, o1, o2, o3):
    ins = (in0, in1, in2, in3)
    outs = (ot0, ot1, ot2, ot3)
    gsems = (g0, g1, g2, g3)
    osems = (o0, o1, o2, o3)

    wid = lax.axis_index("s") * NCORES + lax.axis_index("c")
    row0 = wid * RPW

    pltpu.sync_copy(idx_hbm.at[pl.ds(wid * NCH, NCH)], idx_v)
    pltpu.sync_copy(pos_hbm, pos_v)

    def gather_start(c, b):
        pltpu.async_copy(table_hbm.at[idx_v.at[c]], ins[b], gsems[b])

    def gather_wait(c, b):
        pltpu.make_async_copy(table_hbm.at[idx_v.at[c]], ins[b], gsems[b]).wait()

    def out_start(c, b):
        pltpu.async_copy(outs[b], out_hbm.at[pl.ds(row0 + c * CHUNK, CHUNK)],
                         osems[b])

    def out_wait(c, b):
        pltpu.make_async_copy(outs[b],
                              out_hbm.at[pl.ds(row0 + c * CHUNK, CHUNK)],
                              osems[b]).wait()

    def compute(c, b):
        off = lax.rem(c, 5) * CHUNK
        src = ins[b]
        dst = outs[b]

        @plsc.parallel_loop(0, CHUNK, unroll=2)
        def row_fma(r):
            p = off + r
            for k in range(DIM // 16):
                sl = pl.ds(k * 16, 16)
                dst[r, sl] = src[r, sl] * SCALE + pos_v[p, sl]

    for b in range(NBUF):
        gather_start(b, b)

    for b in range(NBUF):
        gather_wait(b, b)
        compute(b, b)
        gather_start(b + NBUF, b)
        out_start(b, b)

    def cycle(t, _):
        for b in range(NBUF):
            c = t * NBUF + b
            gather_wait(c, b)
            out_wait(c - NBUF, b)
            compute(c, b)
            gather_start(c + NBUF, b)
            out_start(c, b)
        return _

    lax.fori_loop(1, NCH // NBUF - 1, cycle, 0)

    for b in range(NBUF):
        c = NCH - NBUF + b
        gather_wait(c, b)
        out_wait(c - NBUF, b)
        compute(c, b)
        out_start(c, b)
    for b in range(NBUF):
        out_wait(NCH - NBUF + b, b)


def kernel(inputs, table, pos_enc):
    flat_idx = inputs.reshape(ROWS // CHUNK, CHUNK).astype(jnp.int32)

    mesh = plsc.VectorSubcoreMesh(core_axis_name="c", subcore_axis_name="s")
    run = pl.kernel(
        _body,
        mesh=mesh,
        out_type=jax.ShapeDtypeStruct((ROWS, DIM), jnp.float32),
        scratch_types=[
            pltpu.VMEM((NCH, CHUNK), jnp.int32),
            pltpu.VMEM((SEQ, DIM), jnp.float32),
        ] + [pltpu.VMEM((CHUNK, DIM), jnp.float32)] * (2 * NBUF)
          + [pltpu.SemaphoreType.DMA] * (2 * NBUF),
    )
    out = run(flat_idx, table, pos_enc)
    return out.reshape(1024, SEQ, DIM)


# speedup vs baseline: 6.9613x; 1.0035x over previous
import jax
import jax.numpy as jnp
from jax import lax
from jax.experimental import pallas as pl
from jax.experimental.pallas import tpu as pltpu
from jax.experimental.pallas import tpu_sc as plsc

SEQ = 200
DIM = 128
NCORES = 2
NSUB = 16
NW = NCORES * NSUB
ROWS = 1024 * SEQ
RPW = ROWS // NW
CHUNK = 40
NCH = RPW // CHUNK
NBUF = 8
SCALE = 11.313708498984761


def _body(idx_hbm, table_hbm, pos_hbm, out_hbm, idx_v, pos_v, *rest):
    ins = rest[:NBUF]
    outs = rest[NBUF:2 * NBUF]
    gsems = rest[2 * NBUF:3 * NBUF]
    osems = rest[3 * NBUF:4 * NBUF]

    wid = lax.axis_index("s") * NCORES + lax.axis_index("c")
    row0 = wid * RPW

    pltpu.sync_copy(idx_hbm.at[pl.ds(wid * NCH, NCH)], idx_v)

    def gather_start(c, b):
        pltpu.async_copy(table_hbm.at[idx_v.at[c]], ins[b], gsems[b])

    def gather_wait(c, b):
        pltpu.make_async_copy(table_hbm.at[idx_v.at[c]], ins[b], gsems[b]).wait()

    def out_start(c, b):
        pltpu.async_copy(outs[b], out_hbm.at[pl.ds(row0 + c * CHUNK, CHUNK)],
                         osems[b])

    def out_wait(c, b):
        pltpu.make_async_copy(outs[b],
                              out_hbm.at[pl.ds(row0 + c * CHUNK, CHUNK)],
                              osems[b]).wait()

    def compute(c, b):
        off = lax.rem(c, 5) * CHUNK
        src = ins[b]
        dst = outs[b]

        @plsc.parallel_loop(0, CHUNK, unroll=2)
        def row_fma(r):
            p = off + r
            for k in range(DIM // 16):
                sl = pl.ds(k * 16, 16)
                dst[r, sl] = src[r, sl] * SCALE + pos_v[p, sl]

    for b in range(NBUF):
        gather_start(b, b)
    pltpu.sync_copy(pos_hbm, pos_v)

    for b in range(NBUF):
        gather_wait(b, b)
        compute(b, b)
        gather_start(b + NBUF, b)
        out_start(b, b)

    def cycle(t, _):
        for b in range(NBUF):
            c = t * NBUF + b
            gather_wait(c, b)
            out_wait(c - NBUF, b)
            compute(c, b)
            gather_start(c + NBUF, b)
            out_start(c, b)
        return _

    lax.fori_loop(1, NCH // NBUF - 1, cycle, 0)

    for b in range(NBUF):
        c = NCH - NBUF + b
        gather_wait(c, b)
        out_wait(c - NBUF, b)
        compute(c, b)
        out_start(c, b)
    for b in range(NBUF):
        out_wait(NCH - NBUF + b, b)


def kernel(inputs, table, pos_enc):
    flat_idx = inputs.reshape(ROWS // CHUNK, CHUNK).astype(jnp.int32)

    mesh = plsc.VectorSubcoreMesh(core_axis_name="c", subcore_axis_name="s")
    run = pl.kernel(
        _body,
        mesh=mesh,
        out_type=jax.ShapeDtypeStruct((ROWS, DIM), jnp.float32),
        scratch_types=[
            pltpu.VMEM((NCH, CHUNK), jnp.int32),
            pltpu.VMEM((SEQ, DIM), jnp.float32),
        ] + [pltpu.VMEM((CHUNK, DIM), jnp.float32)] * (2 * NBUF)
          + [pltpu.SemaphoreType.DMA] * (2 * NBUF),
    )
    out = run(flat_idx, table, pos_enc)
    return out.reshape(1024, SEQ, DIM)
